# baseline (speedup 1.0000x reference)
import functools

import jax
import jax.numpy as jnp
from jax import lax
from jax.experimental import pallas as pl
from jax.experimental.pallas import tpu as pltpu
from jax.experimental.pallas import tpu_sc as plsc

H = 128
NC, NS = 2, 16
NW = NC * NS
CH = 128
EDGE_ALIGN = NW * 2 * CH


def _rup(n, m):
    return ((n + m - 1) // m) * m


def _silu(v):
    return v * jax.nn.sigmoid(v)


def _pad_rows(a, n):
    return jnp.pad(a, ((0, n - a.shape[0]),) + ((0, 0),) * (a.ndim - 1))


@functools.cache
def _gather_pair(Na, Da, Nb, Db, EP):
    PW = EP // NW
    NCH = PW // CH
    PAIRS, ODD = NCH // 2, NCH % 2
    mesh = plsc.VectorSubcoreMesh(core_axis_name="c", subcore_axis_name="s", num_cores=NC, num_subcores=NS)

    def body(a_hbm, b_hbm, ia_hbm, ib_hbm, o1, o2, idxa, idxb, bufa, bufb,
             sga0, sga1, sgb0, sgb1, swa0, swa1, swb0, swb1):
        wid = lax.axis_index("s") * NC + lax.axis_index("c")
        base = wid * PW
        pltpu.sync_copy(ia_hbm.at[pl.ds(base, PW)], idxa)
        pltpu.sync_copy(ib_hbm.at[pl.ds(base, PW)], idxb)

        def fire(j, ba, sga, sgb):
            ga = pltpu.async_copy(a_hbm.at[idxa.at[pl.ds(j * CH, CH)]],
                                  bufa.at[ba], sga)
            gb = pltpu.async_copy(b_hbm.at[idxb.at[pl.ds(j * CH, CH)]],
                                  bufb.at[ba], sgb)
            return ga, gb

        def wback(j, ba, swa, swb):
            pltpu.async_copy(bufa.at[ba], o1.at[pl.ds(base + j * CH, CH), :], swa)
            pltpu.async_copy(bufb.at[ba], o2.at[pl.ds(base + j * CH, CH), :], swb)

        def drain(ba, swa, swb):
            pltpu.make_async_copy(bufa.at[ba], o1.at[pl.ds(0, CH), :], swa).wait()
            pltpu.make_async_copy(bufb.at[ba], o2.at[pl.ds(0, CH), :], swb).wait()

        def do_pair(t, carry):
            j0 = t * 2

            @pl.when(t > 0)
            def _():
                drain(0, swa0, swb0)
                drain(1, swa1, swb1)

            ga0, gb0 = fire(j0, 0, sga0, sgb0)
            ga1, gb1 = fire(j0 + 1, 1, sga1, sgb1)
            ga0.wait()
            gb0.wait()
            wback(j0, 0, swa0, swb0)
            ga1.wait()
            gb1.wait()
            wback(j0 + 1, 1, swa1, swb1)
            return carry

        lax.fori_loop(0, PAIRS, do_pair, 0)
        if ODD:
            if PAIRS:
                drain(0, swa0, swb0)
            ga0, gb0 = fire(NCH - 1, 0, sga0, sgb0)
            ga0.wait()
            gb0.wait()
            wback(NCH - 1, 0, swa0, swb0)
        if PAIRS:
            drain(1, swa1, swb1)
        if PAIRS or ODD:
            drain(0, swa0, swb0)

    return pl.kernel(
        body,
        out_type=(jax.ShapeDtypeStruct((EP, Da), jnp.float32),
                  jax.ShapeDtypeStruct((EP, Db), jnp.float32)),
        mesh=mesh,
        scratch_types=[pltpu.VMEM((PW,), jnp.int32),
                       pltpu.VMEM((PW,), jnp.int32),
                       pltpu.VMEM((2, CH, Da), jnp.float32),
                       pltpu.VMEM((2, CH, Db), jnp.float32)]
                      + [pltpu.SemaphoreType.DMA] * 8,
    )


@functools.cache
def _scatter(sets, RP):
    DUMP_ROWS = 8
    AR = RP + DUMP_ROWS
    TPR = RP // NS
    mesh = plsc.VectorSubcoreMesh(core_axis_name="c", subcore_axis_name="s", num_cores=NC, num_subcores=NS)
    specs = []
    for (EP, split) in sets:
        EC = EP // 2 if split else EP
        CB = EC if split else 0
        ECT = EC // NS
        specs.append((EP, EC, CB, ECT, ECT // CH))
    TK = sum(k for *_, k in specs)
    nset = len(sets)

    def body(*refs):
        zero_hbm = refs[0]
        rows_hbms = refs[1:1 + nset]
        lidx_hbms = refs[1 + nset:1 + 2 * nset]
        out = refs[1 + 2 * nset]
        lidxv, buf, acc, sl0, sl1, ss0, ss1 = refs[2 + 2 * nset:]
        c = lax.axis_index("c")
        s = lax.axis_index("s")
        pltpu.sync_copy(zero_hbm, buf.at[0])
        for z in range(TPR // CH):
            pltpu.sync_copy(buf.at[0], acc.at[pl.ds(s * TPR + z * CH, CH), :])
        plsc.subcore_barrier()

        off = 0
        for (rows_hbm, lidx_hbm, (EP, EC, CB, ECT, K)) in zip(rows_hbms, lidx_hbms, specs):
            pltpu.sync_copy(lidx_hbm.at[c, s], lidxv.at[pl.ds(off, K), :])
            ebase = c * CB + s * ECT

            def pairbody(t, carry, rows_hbm=rows_hbm, ebase=ebase, off=off):
                j0 = t * 2

                @pl.when(t > 0)
                def _():
                    pltpu.make_async_copy(buf.at[0], acc.at[lidxv.at[off]],
                                          ss0).wait()
                    pltpu.make_async_copy(buf.at[1], acc.at[lidxv.at[off]],
                                          ss1).wait()

                l0 = pltpu.async_copy(
                    rows_hbm.at[pl.ds(ebase + j0 * CH, CH), :], buf.at[0], sl0)
                l1 = pltpu.async_copy(
                    rows_hbm.at[pl.ds(ebase + (j0 + 1) * CH, CH), :], buf.at[1], sl1)
                l0.wait()
                pltpu.async_copy(buf.at[0], acc.at[lidxv.at[off + j0]],
                                 ss0, add=True)
                l1.wait()
                pltpu.async_copy(buf.at[1], acc.at[lidxv.at[off + j0 + 1]],
                                 ss1, add=True)
                return carry

            lax.fori_loop(0, K // 2, pairbody, 0)
            pltpu.make_async_copy(buf.at[0], acc.at[lidxv.at[off]], ss0).wait()
            pltpu.make_async_copy(buf.at[1], acc.at[lidxv.at[off]], ss1).wait()
            off += K

        plsc.subcore_barrier()
        pltpu.sync_copy(acc.at[pl.ds(s * TPR, TPR), :],
                        out.at[c, pl.ds(s * TPR, TPR), :])

    return pl.kernel(
        body,
        out_type=jax.ShapeDtypeStruct((2, RP, H), jnp.float32),
        mesh=mesh,
        scratch_types=[pltpu.VMEM((TK, CH), jnp.int32),
                       pltpu.VMEM((2, CH, H), jnp.float32),
                       pltpu.VMEM_SHARED((AR, H), jnp.float32)]
                      + [pltpu.SemaphoreType.DMA] * 4,
    )


def _rspec(R, D=H):
    return pl.BlockSpec((R, D), lambda i: (i, 0))


def _wspec(shape):
    return pl.BlockSpec(shape, lambda i: tuple(0 for _ in shape))


@functools.cache
def _node_base_kernel(NP, nin, n_agg, scale, residual, n_proj, pre):
    R = 256
    grid = (NP // R,)
    n_w = (5 if n_agg else 2) + (4 if pre else 2 * n_proj)
    n_out = 1 if pre else 1 + n_proj

    def body(*refs):
        ins = refs[:nin]
        aggs = refs[nin:nin + n_agg]
        wrefs = refs[nin + n_agg:nin + n_agg + n_w]
        outs = refs[nin + n_agg + n_w:]
        hsum = ins[0][...]
        for r in ins[1:]:
            hsum = hsum + r[...]
        if n_agg:
            wh, wa, b1, w2, b2 = (w[...] for w in wrefs[:5])
            agg = aggs[0][...]
            for r in aggs[1:]:
                agg = agg + r[...]
            t = _silu(jnp.dot(hsum, wh, preferred_element_type=jnp.float32)
                      + jnp.dot(agg, wa, preferred_element_type=jnp.float32) + b1)
            base = hsum + jnp.dot(t, w2, preferred_element_type=jnp.float32) + b2
            ew = wrefs[5:]
        else:
            w, b = wrefs[0][...], wrefs[1][...]
            base = jnp.dot(hsum * scale, w, preferred_element_type=jnp.float32) + b
            ew = wrefs[2:]
        if pre:
            p1, pb1, p2, pb2 = (w[...] for w in ew)
            t = _silu(jnp.dot(base, p1, preferred_element_type=jnp.float32) + pb1)
            outs[0][...] = jnp.dot(t, p2, preferred_element_type=jnp.float32) + pb2
        else:
            outs[0][...] = base
            for k in range(n_proj):
                outs[1 + k][...] = (
                    jnp.dot(base, ew[2 * k][...],
                            preferred_element_type=jnp.float32) + ew[2 * k + 1][...])

    if n_agg:
        wsp = [_wspec((H, H)), _wspec((H, H)), _wspec((1, H)), _wspec((H, H)),
               _wspec((1, H))]
    else:
        wsp = [_wspec((H, H)), _wspec((1, H))]
    if pre:
        wsp += [_wspec((H, H)), _wspec((1, H)), _wspec((H, H)), _wspec((1, H))]
    else:
        wsp += [_wspec((H, H)), _wspec((1, H))] * n_proj
    in_specs = [_rspec(R)] * (nin + n_agg) + wsp
    return pl.pallas_call(
        body, grid=grid, in_specs=in_specs,
        out_specs=[_rspec(R)] * n_out,
        out_shape=[jax.ShapeDtypeStruct((NP, H), jnp.float32)] * n_out,
    )


@functools.cache
def _edge_mlp_kernel(EP):
    R = 512
    grid = (EP // R,)

    def body(gs, gr, inv8, w1inv, w2, b2, winfb, binfb, out):
        g = gs[...] + gr[...] + jnp.dot(inv8[...], w1inv[...],
                                        preferred_element_type=jnp.float32)
        t = _silu(g)
        m = _silu(jnp.dot(t, w2[...], preferred_element_type=jnp.float32) + b2[...])
        w = jax.nn.sigmoid(jnp.dot(m, winfb[...],
                                   preferred_element_type=jnp.float32) + binfb[...])
        out[...] = m * w

    return pl.pallas_call(
        body, grid=grid,
        in_specs=[_rspec(R), _rspec(R), _rspec(R, 8), _wspec((8, H)),
                  _wspec((H, H)), _wspec((1, H)), _wspec((H, H)), _wspec((1, H))],
        out_specs=_rspec(R),
        out_shape=jax.ShapeDtypeStruct((EP, H), jnp.float32),
    )


@functools.cache
def _inv_kernel(EP):
    R = 512
    grid = (EP // R,)

    def body(cs, cr, out):
        a, b = cs[...], cr[...]
        d = a - b
        n1 = jnp.sqrt(jnp.sum(d * d, axis=1, keepdims=True))
        n2 = jnp.sqrt(jnp.sum(a * a, axis=1, keepdims=True))
        n3 = jnp.sqrt(jnp.sum(b * b, axis=1, keepdims=True))
        out[...] = jnp.concatenate(
            [n1, n2, n3, jnp.zeros((R, 5), jnp.float32)], axis=1)

    return pl.pallas_call(
        body, grid=grid,
        in_specs=[_rspec(R), _rspec(R)],
        out_specs=_rspec(R, 8),
        out_shape=jax.ShapeDtypeStruct((EP, 8), jnp.float32),
    )


@functools.cache
def _avg2_kernel(NP, D):
    R = 512
    grid = (NP // R,)

    def body(a, b, out):
        out[...] = (a[...] + b[...]) * 0.5

    return pl.pallas_call(
        body, grid=grid,
        in_specs=[_rspec(R, D), _rspec(R, D)],
        out_specs=_rspec(R, D),
        out_shape=jax.ShapeDtypeStruct((NP, D), jnp.float32),
    )


@functools.cache
def _pool_kernel(NP, B):
    R = 256
    grid = (NP // R,)

    def body(f, bt, out):
        i = pl.program_id(0)

        @pl.when(i == 0)
        def _():
            out[...] = jnp.zeros((B, H), jnp.float32)

        bids = bt[0, 0, :]
        sel = (bids[None, :] == lax.broadcasted_iota(jnp.int32, (B, R), 0)
               ).astype(jnp.float32)
        out[...] += jnp.dot(sel, f[...], preferred_element_type=jnp.float32)

    return pl.pallas_call(
        body, grid=grid,
        in_specs=[_rspec(R), pl.BlockSpec((1, 1, R), lambda i: (i, 0, 0))],
        out_specs=_wspec((B, H)),
        out_shape=jax.ShapeDtypeStruct((B, H), jnp.float32),
    )


@functools.cache
def _post_kernel(B):
    def body(p0, p1, a0, a1, b, w2b, b2b, out):
        t = _silu(jnp.dot(p0[...], a0[...], preferred_element_type=jnp.float32)
                  + jnp.dot(p1[...], a1[...], preferred_element_type=jnp.float32)
                  + b[...])
        out[...] = jnp.dot(t, w2b[...], preferred_element_type=jnp.float32) + b2b[...]

    return pl.pallas_call(
        body, grid=(1,),
        in_specs=[_wspec((B, H)), _wspec((B, H)), _wspec((H, H)), _wspec((H, H)),
                  _wspec((1, H)), _wspec((H, H)), _wspec((1, H))],
        out_specs=_wspec((B, H)),
        out_shape=jax.ShapeDtypeStruct((B, H), jnp.float32),
    )


def _pad_idx(i, ep):
    return jnp.concatenate([i, jnp.zeros((ep - i.shape[0],), jnp.int32)])


def kernel(pos, x, x_0, x_1, adj_0_0, adj_0_1, adj_1_1, x_0_batch, x_1_batch,
           y, params):
    N0, N1 = x.shape[0], x_1.shape[0]
    B = y.shape[0]
    N0P, N1P = _rup(N0, 2048), _rup(N1, 2048)
    RP = N0P
    DUMP = RP
    adjs = {"0_0": adj_0_0, "0_1": adj_0_1, "1_1": adj_1_1}
    dims_of = {"0_0": ("0", "0"), "0_1": ("0", "1"), "1_1": ("1", "1")}
    EPs = {a: _rup(adjs[a].shape[1], EDGE_ALIGN) for a in adjs}
    NPd = {"0": N0P, "1": N1P}

    x_pad = _pad_rows(x, N0P)
    pos16 = jnp.pad(pos, ((0, N0P - N0), (0, H - pos.shape[1])))
    v0 = _pad_idx(x_1[:, 0], N1P)
    v1 = _pad_idx(x_1[:, 1], N1P)

    sidx, ridx = {}, {}
    for a in adjs:
        sidx[a] = _pad_idx(adjs[a][0], EPs[a])
        ridx[a] = _pad_idx(adjs[a][1], EPs[a])

    def _lidx_split(r, ep):
        rp = jnp.concatenate([r, jnp.full((ep - r.shape[0],), -1, jnp.int32)])
        l = jnp.where(rp >= 0, rp, DUMP)
        return l.reshape(2, NS, (ep // 2) // (NS * CH), CH)

    def _lidx_dual(r, ep):
        rp = jnp.concatenate([r, jnp.full((ep - r.shape[0],), -1, jnp.int32)])
        ls = []
        for c in range(2):
            g = rp - c * RP
            ok = (rp >= c * RP) & (rp < (c + 1) * RP)
            ls.append(jnp.where(ok, g, DUMP))
        return jnp.stack(ls).reshape(2, NS, ep // (NS * CH), CH)

    lidx00 = _lidx_split(adjs["0_0"][1], EPs["0_0"])
    lidx01 = _lidx_dual(adjs["0_1"][1], EPs["0_1"])
    lidx11 = _lidx_dual(adjs["1_1"][1], EPs["1_1"])
    z128 = jnp.zeros((CH, H), jnp.float32)

    emb_w = params["emb"]["w"]
    emb_b = params["emb"]["b"].reshape(1, H)
    msg, upd = [], []
    for layer in params["layers"]:
        m = {}
        for a in adjs:
            p = layer["msg"][a]
            w1 = p["l1"]["w"]
            m[a] = dict(
                ws=w1[:H], wr=w1[H:2 * H], b1=p["l1"]["b"].reshape(1, H),
                winv=jnp.pad(w1[2 * H:], ((0, 5), (0, 0))),
                w2=p["l2"]["w"], b2=p["l2"]["b"].reshape(1, H),
                winfb=jnp.broadcast_to(p["inf"]["w"], (H, H)),
                binfb=jnp.broadcast_to(p["inf"]["b"].reshape(1, 1), (1, H)))
        u = {}
        for d in ("0", "1"):
            p = layer["upd"][d]
            w1 = p["l1"]["w"]
            u[d] = (w1[:H], w1[H:], p["l1"]["b"].reshape(1, H),
                    p["l2"]["w"], p["l2"]["b"].reshape(1, H))
        msg.append(m)
        upd.append(u)
    zb = jnp.zeros((1, H), jnp.float32)

    def proj_w(l, d):
        m = msg[l]
        if d == "0":
            return (m["0_0"]["ws"], zb, m["0_0"]["wr"], m["0_0"]["b1"],
                    m["0_1"]["ws"], zb)
        return (m["0_1"]["wr"], m["0_1"]["b1"], m["1_1"]["ws"], zb,
                m["1_1"]["wr"], m["1_1"]["b1"])

    xa, xb = _gather_pair(N0P, H, N0P, H, N1P)(x_pad, x_pad, v0, v1)
    pa, pb = _gather_pair(N0P, H, N0P, H, N1P)(pos16, pos16, v0, v1)
    cpos1 = _avg2_kernel(N1P, H)(pa, pb)
    cpos = {"0": pos16, "1": cpos1}

    inv8 = {}
    for a in adjs:
        da, db = dims_of[a]
        cs, cr = _gather_pair(NPd[da], H, NPd[db], H, EPs[a])(
            cpos[da], cpos[db], sidx[a], ridx[a])
        inv8[a] = _inv_kernel(EPs[a])(cs, cr)

    h0, *P0 = _node_base_kernel(N0P, 1, 0, 1.0, False, 3, False)(
        x_pad, emb_w, emb_b, *proj_w(0, "0"))
    h1, *P1 = _node_base_kernel(N1P, 2, 0, 0.5, False, 3, False)(
        xa, xb, emb_w, emb_b, *proj_w(0, "1"))

    for l in range(4):
        tabs = {"0_0": (P0[0], P0[1]), "0_1": (P0[2], P1[0]),
                "1_1": (P1[1], P1[2])}
        outs = {}
        for a in adjs:
            da, db = dims_of[a]
            gs, gr = _gather_pair(NPd[da], H, NPd[db], H, EPs[a])(
                tabs[a][0], tabs[a][1], sidx[a], ridx[a])
            mw = msg[l][a]
            outs[a] = _edge_mlp_kernel(EPs[a])(
                gs, gr, inv8[a], mw["winv"], mw["w2"], mw["b2"],
                mw["winfb"], mw["binfb"])
        agg0p = _scatter(((EPs["0_0"], True),), RP)(z128, outs["0_0"], lidx00)
        agg1p = _scatter(((EPs["0_1"], False), (EPs["1_1"], False)), RP)(
            z128, outs["0_1"], outs["1_1"], lidx01, lidx11)
        agg1 = agg1p.reshape(N1P, H)
        u0, u1 = upd[l]["0"], upd[l]["1"]
        if l < 3:
            h0, *P0 = _node_base_kernel(N0P, 1, 2, 1.0, True, 3, False)(
                h0, agg0p[0], agg0p[1], *u0, *proj_w(l + 1, "0"))
            h1, *P1 = _node_base_kernel(N1P, 1, 1, 1.0, True, 3, False)(
                h1, agg1, *u1, *proj_w(l + 1, "1"))
        else:
            pre0 = params["pre"]["0"]
            pre1 = params["pre"]["1"]
            (f0,) = _node_base_kernel(N0P, 1, 2, 1.0, True, 0, True)(
                h0, agg0p[0], agg0p[1], *u0,
                pre0["l1"]["w"], pre0["l1"]["b"].reshape(1, H),
                pre0["l2"]["w"], pre0["l2"]["b"].reshape(1, H))
            (f1,) = _node_base_kernel(N1P, 1, 1, 1.0, True, 0, True)(
                h1, agg1, *u1,
                pre1["l1"]["w"], pre1["l1"]["b"].reshape(1, H),
                pre1["l2"]["w"], pre1["l2"]["b"].reshape(1, H))

    bt0 = jnp.concatenate([x_0_batch, jnp.full((N0P - N0,), -1, jnp.int32)])
    bt1 = jnp.concatenate([x_1_batch, jnp.full((N1P - N1,), -1, jnp.int32)])
    pool0 = _pool_kernel(N0P, B)(f0, bt0.reshape(N0P // 256, 1, 256))
    pool1 = _pool_kernel(N1P, B)(f1, bt1.reshape(N1P // 256, 1, 256))
    pw = params["post"]["l1"]["w"]
    out = _post_kernel(B)(
        pool0, pool1, pw[:H], pw[H:], params["post"]["l1"]["b"].reshape(1, H),
        jnp.broadcast_to(params["post"]["l2"]["w"], (H, H)),
        jnp.broadcast_to(params["post"]["l2"]["b"].reshape(1, 1), (1, H)))
    return out[:, 0]


# speedup vs baseline: 1.0301x; 1.0301x over previous
import functools

import jax
import jax.numpy as jnp
from jax import lax
from jax.experimental import pallas as pl
from jax.experimental.pallas import tpu as pltpu
from jax.experimental.pallas import tpu_sc as plsc

H = 128
NC, NS = 2, 16
NW = NC * NS
CH = 32
NB = 8
NBS = 4
EDGE_ALIGN = 8192


def _rup(n, m):
    return ((n + m - 1) // m) * m


def _silu(v):
    return v * jax.nn.sigmoid(v)


def _pad_rows(a, n):
    return jnp.pad(a, ((0, n - a.shape[0]),) + ((0, 0),) * (a.ndim - 1))


@functools.cache
def _gather_pair(Na, Da, Nb, Db, EP):
    PW = EP // NW
    NCH = PW // CH
    T, REM = NCH // NB, NCH % NB
    mesh = plsc.VectorSubcoreMesh(core_axis_name="c", subcore_axis_name="s", num_cores=NC, num_subcores=NS)

    def body(a_hbm, b_hbm, ia_hbm, ib_hbm, o1, o2, idxa, idxb, bufa, bufb,
             *sems):
        sg, sw = sems[:NB], sems[NB:]
        wid = lax.axis_index("s") * NC + lax.axis_index("c")
        base = wid * PW
        pltpu.sync_copy(ia_hbm.at[pl.ds(base, PW)], idxa)
        pltpu.sync_copy(ib_hbm.at[pl.ds(base, PW)], idxb)

        def fire(j, k):
            ga = pltpu.async_copy(a_hbm.at[idxa.at[pl.ds(j * CH, CH)]],
                                  bufa.at[k], sg[k])
            gb = pltpu.async_copy(b_hbm.at[idxb.at[pl.ds(j * CH, CH)]],
                                  bufb.at[k], sg[k])
            return ga, gb

        def wback(j, k):
            pltpu.async_copy(bufa.at[k], o1.at[pl.ds(base + j * CH, CH), :], sw[k])
            pltpu.async_copy(bufb.at[k], o2.at[pl.ds(base + j * CH, CH), :], sw[k])

        def drain(k):
            pltpu.make_async_copy(bufa.at[k], o1.at[pl.ds(0, CH), :], sw[k]).wait()
            pltpu.make_async_copy(bufb.at[k], o2.at[pl.ds(0, CH), :], sw[k]).wait()

        def block(t, carry):
            j0 = t * NB
            descs = []
            for k in range(NB):
                @pl.when(t > 0)
                def _(k=k):
                    drain(k)
                descs.append(fire(j0 + k, k))
            for k in range(NB):
                ga, gb = descs[k]
                ga.wait()
                gb.wait()
                wback(j0 + k, k)
            return carry

        if T:
            lax.fori_loop(0, T, block, 0)
        descs = []
        for k in range(REM):
            if T:
                drain(k)
            descs.append(fire(T * NB + k, k))
        for k in range(REM):
            ga, gb = descs[k]
            ga.wait()
            gb.wait()
            wback(T * NB + k, k)
        for k in range(REM, NB if T else 0):
            drain(k)
        for k in range(REM):
            drain(k)

    return pl.kernel(
        body,
        out_type=(jax.ShapeDtypeStruct((EP, Da), jnp.float32),
                  jax.ShapeDtypeStruct((EP, Db), jnp.float32)),
        mesh=mesh,
        scratch_types=[pltpu.VMEM((PW,), jnp.int32),
                       pltpu.VMEM((PW,), jnp.int32),
                       pltpu.VMEM((NB, CH, Da), jnp.float32),
                       pltpu.VMEM((NB, CH, Db), jnp.float32)]
                      + [pltpu.SemaphoreType.DMA] * (2 * NB),
    )


@functools.cache
def _scatter(sets, RP):
    DUMP_ROWS = 8
    AR = RP + DUMP_ROWS
    TPR = RP // NS
    mesh = plsc.VectorSubcoreMesh(core_axis_name="c", subcore_axis_name="s", num_cores=NC, num_subcores=NS)
    specs = []
    for (EP, split) in sets:
        EC = EP // 2 if split else EP
        CB = EC if split else 0
        ECT = EC // NS
        specs.append((EP, EC, CB, ECT, ECT // CH))
    TK = sum(k for *_, k in specs)
    nset = len(sets)

    def body(*refs):
        zero_hbm = refs[0]
        rows_hbms = refs[1:1 + nset]
        lidx_hbms = refs[1 + nset:1 + 2 * nset]
        out = refs[1 + 2 * nset]
        lidxv, buf, acc = refs[2 + 2 * nset:5 + 2 * nset]
        sems = refs[5 + 2 * nset:]
        sl, ss, slx = sems[:NBS], sems[NBS:2 * NBS], sems[2 * NBS:]
        c = lax.axis_index("c")
        s = lax.axis_index("s")
        pltpu.sync_copy(zero_hbm, buf.at[0])
        for z in range(TPR // CH):
            pltpu.sync_copy(buf.at[0], acc.at[pl.ds(s * TPR + z * CH, CH), :])
        plsc.subcore_barrier()

        for (rows_hbm, lidx_hbm, (EP, EC, CB, ECT, K)) in zip(rows_hbms, lidx_hbms, specs):
            ebase = c * CB + s * ECT
            nblk = K // NBS

            def lfire(t, slot, lidx_hbm=lidx_hbm):
                return pltpu.async_copy(lidx_hbm.at[c, s, t], lidxv.at[slot],
                                        slx[slot])

            def lwait(slot, lidx_hbm=lidx_hbm):
                pltpu.make_async_copy(lidx_hbm.at[c, s, 0], lidxv.at[slot],
                                      slx[slot]).wait()

            lfire(0, 0)

            def halfblock(t, slot, rows_hbm, ebase):
                j0 = t * NBS
                for k in range(NBS):
                    @pl.when(t > 0)
                    def _(k=k):
                        pltpu.make_async_copy(
                            buf.at[k], acc.at[lidxv.at[0, 0]], ss[k]).wait()

                @pl.when(t + 1 < nblk)
                def _():
                    lfire(t + 1, 1 - slot)

                lwait(slot)
                descs = []
                for k in range(NBS):
                    descs.append(pltpu.async_copy(
                        rows_hbm.at[pl.ds(ebase + (j0 + k) * CH, CH), :],
                        buf.at[k], sl[k]))
                for k in range(NBS):
                    descs[k].wait()
                    pltpu.async_copy(buf.at[k], acc.at[lidxv.at[slot, k]],
                                     ss[k], add=True)

            def blockpair(u, carry, rows_hbm=rows_hbm, ebase=ebase):
                halfblock(u * 2, 0, rows_hbm, ebase)
                halfblock(u * 2 + 1, 1, rows_hbm, ebase)
                return carry

            lax.fori_loop(0, nblk // 2, blockpair, 0)
            for k in range(NBS):
                pltpu.make_async_copy(buf.at[k], acc.at[lidxv.at[0, 0]],
                                      ss[k]).wait()

        plsc.subcore_barrier()
        pltpu.sync_copy(acc.at[pl.ds(s * TPR, TPR), :],
                        out.at[c, pl.ds(s * TPR, TPR), :])

    return pl.kernel(
        body,
        out_type=jax.ShapeDtypeStruct((2, RP, H), jnp.float32),
        mesh=mesh,
        scratch_types=[pltpu.VMEM((2, NBS, CH), jnp.int32),
                       pltpu.VMEM((NBS, CH, H), jnp.float32),
                       pltpu.VMEM_SHARED((AR, H), jnp.float32)]
                      + [pltpu.SemaphoreType.DMA] * (2 * NBS + 2),
    )


def _rspec(R, D=H):
    return pl.BlockSpec((R, D), lambda i: (i, 0))


def _wspec(shape):
    return pl.BlockSpec(shape, lambda i: tuple(0 for _ in shape))


@functools.cache
def _node_base_kernel(NP, nin, n_agg, scale, residual, n_proj, pre):
    R = 256
    grid = (NP // R,)
    n_w = (5 if n_agg else 2) + (4 if pre else 2 * n_proj)
    n_out = 1 if pre else 1 + n_proj

    def body(*refs):
        ins = refs[:nin]
        aggs = refs[nin:nin + n_agg]
        wrefs = refs[nin + n_agg:nin + n_agg + n_w]
        outs = refs[nin + n_agg + n_w:]
        hsum = ins[0][...]
        for r in ins[1:]:
            hsum = hsum + r[...]
        if n_agg:
            wh, wa, b1, w2, b2 = (w[...] for w in wrefs[:5])
            agg = aggs[0][...]
            for r in aggs[1:]:
                agg = agg + r[...]
            t = _silu(jnp.dot(hsum, wh, preferred_element_type=jnp.float32)
                      + jnp.dot(agg, wa, preferred_element_type=jnp.float32) + b1)
            base = hsum + jnp.dot(t, w2, preferred_element_type=jnp.float32) + b2
            ew = wrefs[5:]
        else:
            w, b = wrefs[0][...], wrefs[1][...]
            base = jnp.dot(hsum * scale, w, preferred_element_type=jnp.float32) + b
            ew = wrefs[2:]
        if pre:
            p1, pb1, p2, pb2 = (w[...] for w in ew)
            t = _silu(jnp.dot(base, p1, preferred_element_type=jnp.float32) + pb1)
            outs[0][...] = jnp.dot(t, p2, preferred_element_type=jnp.float32) + pb2
        else:
            outs[0][...] = base
            for k in range(n_proj):
                outs[1 + k][...] = (
                    jnp.dot(base, ew[2 * k][...],
                            preferred_element_type=jnp.float32) + ew[2 * k + 1][...])

    if n_agg:
        wsp = [_wspec((H, H)), _wspec((H, H)), _wspec((1, H)), _wspec((H, H)),
               _wspec((1, H))]
    else:
        wsp = [_wspec((H, H)), _wspec((1, H))]
    if pre:
        wsp += [_wspec((H, H)), _wspec((1, H)), _wspec((H, H)), _wspec((1, H))]
    else:
        wsp += [_wspec((H, H)), _wspec((1, H))] * n_proj
    in_specs = [_rspec(R)] * (nin + n_agg) + wsp
    return pl.pallas_call(
        body, grid=grid, in_specs=in_specs,
        out_specs=[_rspec(R)] * n_out,
        out_shape=[jax.ShapeDtypeStruct((NP, H), jnp.float32)] * n_out,
    )


@functools.cache
def _edge_mlp_kernel(EP):
    R = 512
    grid = (EP // R,)

    def body(gs, gr, inv8, w1inv, w2, b2, winfb, binfb, out):
        g = gs[...] + gr[...] + jnp.dot(inv8[...], w1inv[...],
                                        preferred_element_type=jnp.float32)
        t = _silu(g)
        m = _silu(jnp.dot(t, w2[...], preferred_element_type=jnp.float32) + b2[...])
        w = jax.nn.sigmoid(jnp.dot(m, winfb[...],
                                   preferred_element_type=jnp.float32) + binfb[...])
        out[...] = m * w

    return pl.pallas_call(
        body, grid=grid,
        in_specs=[_rspec(R), _rspec(R), _rspec(R, 8), _wspec((8, H)),
                  _wspec((H, H)), _wspec((1, H)), _wspec((H, H)), _wspec((1, H))],
        out_specs=_rspec(R),
        out_shape=jax.ShapeDtypeStruct((EP, H), jnp.float32),
    )


@functools.cache
def _inv_kernel(EP):
    R = 512
    grid = (EP // R,)

    def body(cs, cr, out):
        a, b = cs[...], cr[...]
        d = a - b
        n1 = jnp.sqrt(jnp.sum(d * d, axis=1, keepdims=True))
        n2 = jnp.sqrt(jnp.sum(a * a, axis=1, keepdims=True))
        n3 = jnp.sqrt(jnp.sum(b * b, axis=1, keepdims=True))
        out[...] = jnp.concatenate(
            [n1, n2, n3, jnp.zeros((R, 5), jnp.float32)], axis=1)

    return pl.pallas_call(
        body, grid=grid,
        in_specs=[_rspec(R), _rspec(R)],
        out_specs=_rspec(R, 8),
        out_shape=jax.ShapeDtypeStruct((EP, 8), jnp.float32),
    )


@functools.cache
def _avg2_kernel(NP, D):
    R = 512
    grid = (NP // R,)

    def body(a, b, out):
        out[...] = (a[...] + b[...]) * 0.5

    return pl.pallas_call(
        body, grid=grid,
        in_specs=[_rspec(R, D), _rspec(R, D)],
        out_specs=_rspec(R, D),
        out_shape=jax.ShapeDtypeStruct((NP, D), jnp.float32),
    )


@functools.cache
def _pool_kernel(NP, B):
    R = 256
    grid = (NP // R,)

    def body(f, bt, out):
        i = pl.program_id(0)

        @pl.when(i == 0)
        def _():
            out[...] = jnp.zeros((B, H), jnp.float32)

        bids = bt[0, 0, :]
        sel = (bids[None, :] == lax.broadcasted_iota(jnp.int32, (B, R), 0)
               ).astype(jnp.float32)
        out[...] += jnp.dot(sel, f[...], preferred_element_type=jnp.float32)

    return pl.pallas_call(
        body, grid=grid,
        in_specs=[_rspec(R), pl.BlockSpec((1, 1, R), lambda i: (i, 0, 0))],
        out_specs=_wspec((B, H)),
        out_shape=jax.ShapeDtypeStruct((B, H), jnp.float32),
    )


@functools.cache
def _post_kernel(B):
    def body(p0, p1, a0, a1, b, w2b, b2b, out):
        t = _silu(jnp.dot(p0[...], a0[...], preferred_element_type=jnp.float32)
                  + jnp.dot(p1[...], a1[...], preferred_element_type=jnp.float32)
                  + b[...])
        out[...] = jnp.dot(t, w2b[...], preferred_element_type=jnp.float32) + b2b[...]

    return pl.pallas_call(
        body, grid=(1,),
        in_specs=[_wspec((B, H)), _wspec((B, H)), _wspec((H, H)), _wspec((H, H)),
                  _wspec((1, H)), _wspec((H, H)), _wspec((1, H))],
        out_specs=_wspec((B, H)),
        out_shape=jax.ShapeDtypeStruct((B, H), jnp.float32),
    )


def _pad_idx(i, ep):
    return jnp.concatenate([i, jnp.zeros((ep - i.shape[0],), jnp.int32)])


def kernel(pos, x, x_0, x_1, adj_0_0, adj_0_1, adj_1_1, x_0_batch, x_1_batch,
           y, params):
    N0, N1 = x.shape[0], x_1.shape[0]
    B = y.shape[0]
    N0P, N1P = _rup(N0, 2048), _rup(N1, 2048)
    RP = N0P
    DUMP = RP
    adjs = {"0_0": adj_0_0, "0_1": adj_0_1, "1_1": adj_1_1}
    dims_of = {"0_0": ("0", "0"), "0_1": ("0", "1"), "1_1": ("1", "1")}
    EPs = {a: _rup(adjs[a].shape[1], EDGE_ALIGN) for a in adjs}
    NPd = {"0": N0P, "1": N1P}

    x_pad = _pad_rows(x, N0P)
    pos16 = jnp.pad(pos, ((0, N0P - N0), (0, H - pos.shape[1])))
    v0 = _pad_idx(x_1[:, 0], N1P)
    v1 = _pad_idx(x_1[:, 1], N1P)

    sidx, ridx = {}, {}
    for a in adjs:
        sidx[a] = _pad_idx(adjs[a][0], EPs[a])
        ridx[a] = _pad_idx(adjs[a][1], EPs[a])

    def _lidx_split(r, ep):
        rp = jnp.concatenate([r, jnp.full((ep - r.shape[0],), -1, jnp.int32)])
        l = jnp.where(rp >= 0, rp, DUMP)
        k = (ep // 2) // (NS * CH)
        return l.reshape(2, NS, k // NBS, NBS, CH)

    def _lidx_dual(r, ep):
        rp = jnp.concatenate([r, jnp.full((ep - r.shape[0],), -1, jnp.int32)])
        ls = []
        for c in range(2):
            g = rp - c * RP
            ok = (rp >= c * RP) & (rp < (c + 1) * RP)
            ls.append(jnp.where(ok, g, DUMP))
        k = ep // (NS * CH)
        return jnp.stack(ls).reshape(2, NS, k // NBS, NBS, CH)

    lidx00 = _lidx_split(adjs["0_0"][1], EPs["0_0"])
    lidx01 = _lidx_dual(adjs["0_1"][1], EPs["0_1"])
    lidx11 = _lidx_dual(adjs["1_1"][1], EPs["1_1"])
    z128 = jnp.zeros((CH, H), jnp.float32)

    emb_w = params["emb"]["w"]
    emb_b = params["emb"]["b"].reshape(1, H)
    msg, upd = [], []
    for layer in params["layers"]:
        m = {}
        for a in adjs:
            p = layer["msg"][a]
            w1 = p["l1"]["w"]
            m[a] = dict(
                ws=w1[:H], wr=w1[H:2 * H], b1=p["l1"]["b"].reshape(1, H),
                winv=jnp.pad(w1[2 * H:], ((0, 5), (0, 0))),
                w2=p["l2"]["w"], b2=p["l2"]["b"].reshape(1, H),
                winfb=jnp.broadcast_to(p["inf"]["w"], (H, H)),
                binfb=jnp.broadcast_to(p["inf"]["b"].reshape(1, 1), (1, H)))
        u = {}
        for d in ("0", "1"):
            p = layer["upd"][d]
            w1 = p["l1"]["w"]
            u[d] = (w1[:H], w1[H:], p["l1"]["b"].reshape(1, H),
                    p["l2"]["w"], p["l2"]["b"].reshape(1, H))
        msg.append(m)
        upd.append(u)
    zb = jnp.zeros((1, H), jnp.float32)

    def proj_w(l, d):
        m = msg[l]
        if d == "0":
            return (m["0_0"]["ws"], zb, m["0_0"]["wr"], m["0_0"]["b1"],
                    m["0_1"]["ws"], zb)
        return (m["0_1"]["wr"], m["0_1"]["b1"], m["1_1"]["ws"], zb,
                m["1_1"]["wr"], m["1_1"]["b1"])

    xa, xb = _gather_pair(N0P, H, N0P, H, N1P)(x_pad, x_pad, v0, v1)
    pa, pb = _gather_pair(N0P, H, N0P, H, N1P)(pos16, pos16, v0, v1)
    cpos1 = _avg2_kernel(N1P, H)(pa, pb)
    cpos = {"0": pos16, "1": cpos1}

    inv8 = {}
    for a in adjs:
        da, db = dims_of[a]
        cs, cr = _gather_pair(NPd[da], H, NPd[db], H, EPs[a])(
            cpos[da], cpos[db], sidx[a], ridx[a])
        inv8[a] = _inv_kernel(EPs[a])(cs, cr)

    h0, *P0 = _node_base_kernel(N0P, 1, 0, 1.0, False, 3, False)(
        x_pad, emb_w, emb_b, *proj_w(0, "0"))
    h1, *P1 = _node_base_kernel(N1P, 2, 0, 0.5, False, 3, False)(
        xa, xb, emb_w, emb_b, *proj_w(0, "1"))

    for l in range(4):
        tabs = {"0_0": (P0[0], P0[1]), "0_1": (P0[2], P1[0]),
                "1_1": (P1[1], P1[2])}
        outs = {}
        for a in adjs:
            da, db = dims_of[a]
            gs, gr = _gather_pair(NPd[da], H, NPd[db], H, EPs[a])(
                tabs[a][0], tabs[a][1], sidx[a], ridx[a])
            mw = msg[l][a]
            outs[a] = _edge_mlp_kernel(EPs[a])(
                gs, gr, inv8[a], mw["winv"], mw["w2"], mw["b2"],
                mw["winfb"], mw["binfb"])
        agg0p = _scatter(((EPs["0_0"], True),), RP)(z128, outs["0_0"], lidx00)
        agg1p = _scatter(((EPs["0_1"], False), (EPs["1_1"], False)), RP)(
            z128, outs["0_1"], outs["1_1"], lidx01, lidx11)
        agg1 = agg1p.reshape(N1P, H)
        u0, u1 = upd[l]["0"], upd[l]["1"]
        if l < 3:
            h0, *P0 = _node_base_kernel(N0P, 1, 2, 1.0, True, 3, False)(
                h0, agg0p[0], agg0p[1], *u0, *proj_w(l + 1, "0"))
            h1, *P1 = _node_base_kernel(N1P, 1, 1, 1.0, True, 3, False)(
                h1, agg1, *u1, *proj_w(l + 1, "1"))
        else:
            pre0 = params["pre"]["0"]
            pre1 = params["pre"]["1"]
            (f0,) = _node_base_kernel(N0P, 1, 2, 1.0, True, 0, True)(
                h0, agg0p[0], agg0p[1], *u0,
                pre0["l1"]["w"], pre0["l1"]["b"].reshape(1, H),
                pre0["l2"]["w"], pre0["l2"]["b"].reshape(1, H))
            (f1,) = _node_base_kernel(N1P, 1, 1, 1.0, True, 0, True)(
                h1, agg1, *u1,
                pre1["l1"]["w"], pre1["l1"]["b"].reshape(1, H),
                pre1["l2"]["w"], pre1["l2"]["b"].reshape(1, H))

    bt0 = jnp.concatenate([x_0_batch, jnp.full((N0P - N0,), -1, jnp.int32)])
    bt1 = jnp.concatenate([x_1_batch, jnp.full((N1P - N1,), -1, jnp.int32)])
    pool0 = _pool_kernel(N0P, B)(f0, bt0.reshape(N0P // 256, 1, 256))
    pool1 = _pool_kernel(N1P, B)(f1, bt1.reshape(N1P // 256, 1, 256))
    pw = params["post"]["l1"]["w"]
    out = _post_kernel(B)(
        pool0, pool1, pw[:H], pw[H:], params["post"]["l1"]["b"].reshape(1, H),
        jnp.broadcast_to(params["post"]["l2"]["w"], (H, H)),
        jnp.broadcast_to(params["post"]["l2"]["b"].reshape(1, 1), (1, H)))
    return out[:, 0]
